# baseline (speedup 1.0000x reference)
import functools

import numpy as np
import jax
import jax.numpy as jnp
from jax import lax
from jax.experimental import pallas as pl
from jax.experimental.pallas import tpu as pltpu
from jax.experimental.pallas import tpu_sc as plsc

_D_MODEL = 64
_CONTEXT_LEN = 200
_NBUF = 2


def _sinusoidal_table(d_model: int, context_len: int) -> np.ndarray:
    pos = np.arange(context_len, dtype=np.float32)[:, None]
    i = np.arange(d_model, dtype=np.float32)[None, :]
    angle_rates = 1.0 / np.power(10000.0, (2.0 * np.floor(i / 2.0)) / float(d_model))
    angles = pos * angle_rates
    table = np.zeros((context_len, d_model), dtype=np.float32)
    table[:, 0::2] = np.sin(angles[:, 0::2])
    table[:, 1::2] = np.cos(angles[:, 1::2])
    return table


def kernel(text_batch, embedding_matrix):
    batch, seq_len = text_batch.shape
    vocab, d_model = embedding_matrix.shape
    assert seq_len == _CONTEXT_LEN and d_model == _D_MODEL

    text_t = text_batch.T
    pos_flat = jnp.asarray(_sinusoidal_table(d_model, seq_len).reshape(-1))

    info = plsc.get_sparse_core_info()
    num_workers = info.num_cores * info.num_subcores
    bpw = batch // num_workers
    assert bpw * num_workers == batch and bpw % 16 == 0
    assert seq_len % _NBUF == 0

    lanes = 16

    mesh = plsc.VectorSubcoreMesh(core_axis_name="c", subcore_axis_name="s")

    @functools.partial(
        pl.kernel,
        mesh=mesh,
        out_type=jax.ShapeDtypeStruct(
            (seq_len, d_model // 8, num_workers, 8, bpw), jnp.float32),
        scratch_types=[
            [pltpu.VMEM((bpw,), jnp.int32) for _ in range(_NBUF)],
            [pltpu.VMEM((bpw, d_model), jnp.float32) for _ in range(_NBUF)],
            [pltpu.VMEM((d_model // 8, 8, bpw + 1), jnp.float32) for _ in range(_NBUF)],
            [pltpu.VMEM((d_model // 8, 8, bpw), jnp.float32) for _ in range(_NBUF)],
            pltpu.VMEM((seq_len * d_model,), jnp.float32),
            [pltpu.SemaphoreType.DMA for _ in range(_NBUF)],
            [pltpu.SemaphoreType.DMA for _ in range(_NBUF)],
            [pltpu.SemaphoreType.DMA for _ in range(_NBUF)],
        ],
        compiler_params=pltpu.CompilerParams(
            use_tc_tiling_on_sc=False, needs_layout_passes=False),
    )
    def _emb_kernel(idx_hbm, table_hbm, pos_hbm, out_hbm,
                    idx_c, gb, tb, tl, pos_v, s_ix, s_g, s_w):
        wid = lax.axis_index("s") * info.num_cores + lax.axis_index("c")
        b0 = wid * bpw
        pltpu.sync_copy(pos_hbm, pos_v)

        def idx_copy(s, k):
            return pltpu.make_async_copy(
                idx_hbm.at[s, pl.ds(b0, bpw)], idx_c[k], s_ix[k])

        def gather_copy(k):
            return pltpu.make_async_copy(table_hbm.at[idx_c[k]], gb[k], s_g[k])

        def write_copy(s, k):
            return pltpu.make_async_copy(
                tl[k], out_hbm.at[s, :, wid, :, :], s_w[k])

        for k in range(_NBUF):
            idx_copy(k, k).start()
        idx_copy(0, 0).wait()
        gather_copy(0).start()

        iota = lax.iota(jnp.int32, lanes)

        def pair_body(i, carry):
            s0 = i * _NBUF
            for b in range(_NBUF):
                s = s0 + b
                bn = (b + 1) % _NBUF

                @pl.when(s + 1 < seq_len)
                def _launch_next_gather():
                    idx_copy(s + 1, bn).wait()
                    gather_copy(bn).start()

                gather_copy(b).wait()

                @pl.when(s + 2 < seq_len)
                def _stage_next_idx():
                    idx_copy(s + 2, b).start()

                @pl.when(s >= _NBUF)
                def _wait_prev_write():
                    write_copy(s - _NBUF, b).wait()

                pvs = [pos_v[pl.ds(s * d_model + c0 * 16, lanes)]
                       for c0 in range(d_model // 16)]
                cbvs = [(c0 * 16 + iota) // 8 for c0 in range(d_model // 16)]
                civs = [(c0 * 16 + iota) % 8 for c0 in range(d_model // 16)]

                def row_body(r, c2, _b=b, _pvs=pvs):
                    rv = jnp.full((lanes,), r, jnp.int32)
                    for c0 in range(d_model // 16):
                        val = gb[_b][r, pl.ds(c0 * 16, lanes)]
                        plsc.store_scatter(
                            tb[_b], [cbvs[c0], civs[c0], rv], val + _pvs[c0])
                    return c2

                lax.fori_loop(0, bpw, row_body, 0, unroll=4)

                def compact_body(j, c2, _b=b):
                    cb8 = j // 8
                    ci8 = j % 8
                    for m in range(bpw // lanes):
                        sl = pl.ds(m * lanes, lanes)
                        tl[_b][cb8, ci8, sl] = tb[_b][cb8, ci8, sl]
                    return c2

                lax.fori_loop(0, (d_model // 8) * 8, compact_body, 0, unroll=4)

                write_copy(s, b).start()
            return carry

        lax.fori_loop(0, seq_len // _NBUF, pair_body, 0)

        for s in range(seq_len - _NBUF, seq_len):
            write_copy(s, s % _NBUF).wait()

    out5 = _emb_kernel(text_t, embedding_matrix, pos_flat)
    return out5.transpose(2, 4, 0, 1, 3).reshape(batch, seq_len, d_model)


# speedup vs baseline: 1.8577x; 1.8577x over previous
import functools

import numpy as np
import jax
import jax.numpy as jnp
from jax import lax
from jax.experimental import pallas as pl
from jax.experimental.pallas import tpu as pltpu
from jax.experimental.pallas import tpu_sc as plsc

_D_MODEL = 64
_CONTEXT_LEN = 200
_NBUF = 4


def _sinusoidal_table(d_model: int, context_len: int) -> np.ndarray:
    pos = np.arange(context_len, dtype=np.float32)[:, None]
    i = np.arange(d_model, dtype=np.float32)[None, :]
    angle_rates = 1.0 / np.power(10000.0, (2.0 * np.floor(i / 2.0)) / float(d_model))
    angles = pos * angle_rates
    table = np.zeros((context_len, d_model), dtype=np.float32)
    table[:, 0::2] = np.sin(angles[:, 0::2])
    table[:, 1::2] = np.cos(angles[:, 1::2])
    return table


def kernel(text_batch, embedding_matrix):
    batch, seq_len = text_batch.shape
    vocab, d_model = embedding_matrix.shape
    assert seq_len == _CONTEXT_LEN and d_model == _D_MODEL

    n_tokens = batch * seq_len
    flat_idx = text_batch.reshape(n_tokens)
    table3 = embedding_matrix.reshape(vocab, 1, d_model)

    info = plsc.get_sparse_core_info()
    num_workers = info.num_cores * info.num_subcores
    per_worker = n_tokens // num_workers
    assert per_worker * num_workers == n_tokens
    chunk = seq_len
    n_chunks = per_worker // chunk
    assert n_chunks * chunk == per_worker and n_chunks % _NBUF == 0

    pos_flat = jnp.asarray(_sinusoidal_table(d_model, seq_len).reshape(-1))

    mesh = plsc.VectorSubcoreMesh(core_axis_name="c", subcore_axis_name="s")

    @functools.partial(
        pl.kernel,
        mesh=mesh,
        out_type=jax.ShapeDtypeStruct((n_tokens, 1, d_model), jnp.float32),
        scratch_types=[
            [pltpu.VMEM((chunk,), jnp.int32) for _ in range(_NBUF)],
            [pltpu.VMEM((chunk, 1, d_model), jnp.float32) for _ in range(_NBUF)],
            pltpu.VMEM((seq_len * d_model,), jnp.float32),
            [pltpu.SemaphoreType.DMA for _ in range(_NBUF)],
            [pltpu.SemaphoreType.DMA for _ in range(_NBUF)],
            [pltpu.SemaphoreType.DMA for _ in range(_NBUF)],
        ],
    )
    def _emb_kernel(idx_hbm, table_hbm, pos_hbm, out_hbm,
                    idx_c, gb, pos_v, s_ix, s_g, s_w):
        wid = lax.axis_index("s") * info.num_cores + lax.axis_index("c")
        base = wid * per_worker
        pltpu.sync_copy(pos_hbm, pos_v)

        def idx_copy(j, k):
            return pltpu.make_async_copy(
                idx_hbm.at[pl.ds(base + j * chunk, chunk)], idx_c[k], s_ix[k])

        def gather_copy(k):
            return pltpu.make_async_copy(table_hbm.at[idx_c[k]], gb[k], s_g[k])

        def write_copy(j, k):
            return pltpu.make_async_copy(
                gb[k], out_hbm.at[pl.ds(base + j * chunk, chunk)], s_w[k])

        for k in range(_NBUF):
            idx_copy(k, k).start()
        for k in range(2):
            idx_copy(k, k).wait()
            gather_copy(k).start()

        def quad_body(i, carry):
            j0 = i * _NBUF
            for b in range(_NBUF):
                j = j0 + b
                bn = (b + 2) % _NBUF

                @pl.when(j + 2 < n_chunks)
                def _launch_next_gather():
                    idx_copy(j + 2, bn).wait()

                    @pl.when(j >= 2)
                    def _wait_prev_write():
                        write_copy(j - 2, bn).wait()

                    gather_copy(bn).start()

                gather_copy(b).wait()

                @pl.when(j + _NBUF < n_chunks)
                def _stage_next_idx():
                    idx_copy(j + _NBUF, b).start()

                def row_body(r, c2):
                    for c in range(d_model // 16):
                        val = pos_v[pl.ds(r * d_model + c * 16, 16)]
                        plsc.addupdate(gb[b].at[r, 0, pl.ds(c * 16, 16)], val)
                    return c2

                lax.fori_loop(0, chunk, row_body, 0, unroll=2)

                write_copy(j, b).start()
            return carry

        lax.fori_loop(0, n_chunks // _NBUF, quad_body, 0)

        for j in range(n_chunks - _NBUF, n_chunks):
            write_copy(j, j % _NBUF).wait()

    out = _emb_kernel(flat_idx, table3, pos_flat)
    return out.reshape(batch, seq_len, d_model)
